# baseline (speedup 1.0000x reference)
import functools

import jax
import jax.numpy as jnp
from jax import lax
from jax.experimental import pallas as pl
from jax.experimental.pallas import tpu as pltpu
from jax.experimental.pallas import tpu_sc as plsc

N_BOXES = 20000
IOU_T = 0.5
B = 1024


CH = 128


def _nms_body(x1r, y1r, x2r, y2r, eye, tru, keep_out,
              llx1, lly1, llx2, lly2, qref, nb):
    j = pl.program_id(0)

    @pl.when(j == 0)
    def _init():
        qref[0] = 0

    def row(ref, i):
        return ref[pl.ds(i, 1)].reshape(1, B)

    def to_col(v_row):
        return lax.dot_general(eye[...], v_row, (((1,), (1,)), ((), ())),
                               precision=lax.Precision.HIGHEST,
                               preferred_element_type=jnp.float32)

    x1jr, y1jr = row(x1r, j), row(y1r, j)
    x2jr, y2jr = row(x2r, j), row(y2r, j)
    x1j, y1j, x2j, y2j = map(to_col, (x1jr, y1jr, x2jr, y2jr))
    aj = jnp.maximum(x2j - x1j, 0.0) * jnp.maximum(y2j - y1j, 0.0)

    def pair_sup(x1i, y1i, x2i, y2i):
        ai = jnp.maximum(x2i - x1i, 0.0) * jnp.maximum(y2i - y1i, 0.0)
        w = jnp.maximum(jnp.minimum(x2i, x2j) - jnp.maximum(x1i, x1j), 0.0)
        h = jnp.maximum(jnp.minimum(y2i, y2j) - jnp.maximum(y1i, y1j), 0.0)
        inter = w * h
        denom = ai + aj - inter + 1e-12
        return inter / denom > IOU_T

    q = qref[0]

    def chunk(ref, i):
        return ref[pl.ds(i, 1)].reshape(1, CH)

    def cross_body(i, sup):
        m = pair_sup(chunk(llx1, i), chunk(lly1, i),
                     chunk(llx2, i), chunk(lly2, i))
        return jnp.maximum(
            sup, jnp.max(m.astype(jnp.float32), axis=1, keepdims=True))

    sup = lax.fori_loop(0, q, cross_body,
                        jnp.zeros((B, 1), dtype=jnp.float32))

    tri = (lax.broadcasted_iota(jnp.int32, (B, B), 1)
           < lax.broadcasted_iota(jnp.int32, (B, B), 0))
    m_intra = (pair_sup(x1jr, y1jr, x2jr, y2jr) & tri).astype(jnp.float32)

    k0 = 1.0 - sup

    def fp_cond(st):
        _, changed, it = st
        return jnp.logical_and(changed > 0, it < B)

    def fp_body(st):
        k, _, it = st
        scol = lax.dot_general(m_intra, k, (((1,), (0,)), ((), ())),
                               precision=lax.Precision.HIGHEST,
                               preferred_element_type=jnp.float32)
        kn = jnp.where(scol > 0.0, 0.0, k0)
        changed = jnp.sum((kn != k).astype(jnp.int32))
        return kn, changed, it + 1

    k, _, _ = lax.while_loop(
        fp_cond, fp_body, (k0, jnp.int32(1), jnp.int32(0)))

    k_row = lax.dot_general(k, eye[...], (((0,), (0,)), ((), ())),
                            precision=lax.Precision.HIGHEST,
                               preferred_element_type=jnp.float32)
    keep_out[pl.ds(j, 1)] = k_row.reshape(1, 1, B)

    c_row = lax.dot_general(k_row, tru[...], (((1,), (0,)), ((), ())),
                            precision=lax.Precision.HIGHEST,
                            preferred_element_type=jnp.float32)
    c_col = lax.dot_general(eye[...], c_row, (((1,), (1,)), ((), ())),
                            precision=lax.Precision.HIGHEST,
                            preferred_element_type=jnp.float32)
    slot_t = lax.broadcasted_iota(jnp.int32, (B, B), 1).astype(jnp.float32)
    perm_t = ((c_col == slot_t + 1.0) & (k > 0.0)).astype(jnp.float32)

    def compact(v_row):
        return lax.dot_general(v_row, perm_t, (((1,), (0,)), ((), ())),
                               precision=lax.Precision.HIGHEST,
                               preferred_element_type=jnp.float32)

    cx1, cy1 = compact(x1jr), compact(y1jr)
    cx2, cy2 = compact(x2jr), compact(y2jr)
    for t in range(B // CH):
        sl = slice(t * CH, (t + 1) * CH)
        llx1[pl.ds(q + t, 1)] = cx1[:, sl].reshape(1, 1, CH)
        lly1[pl.ds(q + t, 1)] = cy1[:, sl].reshape(1, 1, CH)
        llx2[pl.ds(q + t, 1)] = cx2[:, sl].reshape(1, 1, CH)
        lly2[pl.ds(q + t, 1)] = cy2[:, sl].reshape(1, 1, CH)
    c_total = jnp.sum(k).astype(jnp.int32)
    qref[0] = q + (c_total + CH - 1) // CH


def _nms_keep(bp):
    npad, _ = bp.shape
    nb = npad // B
    x1 = bp[:, 0].reshape(nb, 1, B)
    y1 = bp[:, 1].reshape(nb, 1, B)
    x2 = bp[:, 2].reshape(nb, 1, B)
    y2 = bp[:, 3].reshape(nb, 1, B)
    eye = jnp.eye(B, dtype=jnp.float32)
    tru = jnp.triu(jnp.ones((B, B), jnp.float32))
    full_r = pl.BlockSpec((nb, 1, B), lambda j: (0, 0, 0))
    full_e = pl.BlockSpec((B, B), lambda j: (0, 0))
    nchunks = npad // CH
    keep = pl.pallas_call(
        functools.partial(_nms_body, nb=nb),
        grid=(nb,),
        in_specs=[full_r, full_r, full_r, full_r, full_e, full_e],
        out_specs=full_r,
        out_shape=jax.ShapeDtypeStruct((nb, 1, B), jnp.float32),
        scratch_shapes=[pltpu.VMEM((nchunks, 1, CH), jnp.float32)
                        for _ in range(4)]
        + [pltpu.SMEM((1,), jnp.int32)],
    )(x1, y1, x2, y2, eye, tru)
    return keep.reshape(npad)


_SC_NW = 32
_SC_ROWS_PER_W = 640
_SC_CHUNK = 128
_SC_PAD = _SC_NW * _SC_ROWS_PER_W


def _sc_sorted_gather(table, idx):
    mesh = plsc.VectorSubcoreMesh(core_axis_name="c", subcore_axis_name="s")

    @functools.partial(
        pl.kernel, mesh=mesh,
        out_type=jax.ShapeDtypeStruct((_SC_PAD, 128), jnp.float32),
        scratch_types=[
            pltpu.VMEM((_SC_ROWS_PER_W,), jnp.int32),
            pltpu.VMEM((_SC_ROWS_PER_W, 128), jnp.float32),
            pltpu.SemaphoreType.DMA,
        ],
    )
    def gather_k(table_hbm, idx_hbm, out_hbm, idx_v, rows_v, sem):
        wid = lax.axis_index("s") * 2 + lax.axis_index("c")
        base = wid * _SC_ROWS_PER_W
        pltpu.sync_copy(idx_hbm.at[pl.ds(base, _SC_ROWS_PER_W)], idx_v)
        copies = [
            pltpu.async_copy(
                table_hbm.at[idx_v.at[pl.ds(c * _SC_CHUNK, _SC_CHUNK)]],
                rows_v.at[pl.ds(c * _SC_CHUNK, _SC_CHUNK)], sem)
            for c in range(_SC_ROWS_PER_W // _SC_CHUNK)
        ]
        for cp in copies:
            cp.wait()
        pltpu.sync_copy(rows_v, out_hbm.at[pl.ds(base, _SC_ROWS_PER_W)])

    return gather_k(table, idx)


def kernel(boxes, scores):
    n = boxes.shape[0]
    order = jnp.argsort(-scores)
    table = jnp.concatenate(
        [boxes, scores[:, None], jnp.zeros((n, 123), jnp.float32)], axis=1)
    idx = jnp.concatenate(
        [order.astype(jnp.int32),
         jnp.zeros((_SC_PAD - n,), jnp.int32)])
    sorted_rows = _sc_sorted_gather(table, idx)
    b = sorted_rows[:n, :4]
    s = sorted_rows[:n, 4]
    npad = ((n + B - 1) // B) * B
    pad = jnp.tile(jnp.array([[0.0, 0.0, -1.0, -1.0]], jnp.float32),
                   (npad - n, 1))
    bp = jnp.concatenate([b, pad], axis=0)
    keep = _nms_keep(bp)[:n]
    out = jnp.concatenate([b * keep[:, None], (s * keep)[:, None]], axis=1)
    return out


# speedup vs baseline: 1.0934x; 1.0934x over previous
import functools

import jax
import jax.numpy as jnp
from jax import lax
from jax.experimental import pallas as pl
from jax.experimental.pallas import tpu as pltpu
from jax.experimental.pallas import tpu_sc as plsc

N_BOXES = 20000
IOU_T = 0.5
B = 1024


CH = 128


def _nms_body(x1r, y1r, x2r, y2r, eye, tru, keep_out,
              llx1, lly1, llx2, lly2, qref, nb):
    j = pl.program_id(0)

    @pl.when(j == 0)
    def _init():
        qref[0] = 0

    def row(ref, i):
        return ref[pl.ds(i, 1)].reshape(1, B)

    def to_col(v_row):
        return lax.dot_general(eye[...], v_row, (((1,), (1,)), ((), ())),
                               precision=lax.Precision.HIGHEST,
                               preferred_element_type=jnp.float32)

    x1jr, y1jr = row(x1r, j), row(y1r, j)
    x2jr, y2jr = row(x2r, j), row(y2r, j)
    x1j, y1j, x2j, y2j = map(to_col, (x1jr, y1jr, x2jr, y2jr))
    aj = jnp.maximum(x2j - x1j, 0.0) * jnp.maximum(y2j - y1j, 0.0)

    def pair_sup(x1i, y1i, x2i, y2i):
        ai = jnp.maximum(x2i - x1i, 0.0) * jnp.maximum(y2i - y1i, 0.0)
        w = jnp.maximum(jnp.minimum(x2i, x2j) - jnp.maximum(x1i, x1j), 0.0)
        h = jnp.maximum(jnp.minimum(y2i, y2j) - jnp.maximum(y1i, y1j), 0.0)
        inter = w * h
        denom = ai + aj - inter + 1e-12
        return inter / denom > IOU_T

    q = qref[0]
    nch = (q + B - 1) // B

    def cross_body(i, sup):
        m = pair_sup(row(llx1, i), row(lly1, i), row(llx2, i), row(lly2, i))
        return jnp.maximum(
            sup, jnp.max(m.astype(jnp.float32), axis=1, keepdims=True))

    sup = lax.fori_loop(0, nch, cross_body,
                        jnp.zeros((B, 1), dtype=jnp.float32))

    tri = (lax.broadcasted_iota(jnp.int32, (B, B), 1)
           < lax.broadcasted_iota(jnp.int32, (B, B), 0))
    m_intra = (pair_sup(x1jr, y1jr, x2jr, y2jr) & tri).astype(jnp.float32)

    k0 = 1.0 - sup

    def fp_cond(st):
        _, changed, it = st
        return jnp.logical_and(changed > 0, it < B)

    def fp_body(st):
        k, _, it = st
        scol = lax.dot_general(m_intra, k, (((1,), (0,)), ((), ())),
                               precision=lax.Precision.HIGHEST,
                               preferred_element_type=jnp.float32)
        kn = jnp.where(scol > 0.0, 0.0, k0)
        changed = jnp.sum((kn != k).astype(jnp.int32))
        return kn, changed, it + 1

    k, _, _ = lax.while_loop(
        fp_cond, fp_body, (k0, jnp.int32(1), jnp.int32(0)))

    k_row = lax.dot_general(k, eye[...], (((0,), (0,)), ((), ())),
                            precision=lax.Precision.HIGHEST,
                               preferred_element_type=jnp.float32)
    keep_out[pl.ds(j, 1)] = k_row.reshape(1, 1, B)

    c_row = lax.dot_general(k_row, tru[...], (((1,), (0,)), ((), ())),
                            precision=lax.Precision.HIGHEST,
                            preferred_element_type=jnp.float32)
    c_col = lax.dot_general(eye[...], c_row, (((1,), (1,)), ((), ())),
                            precision=lax.Precision.HIGHEST,
                            preferred_element_type=jnp.float32)
    qc = q // B
    off = q - qc * B
    pos = c_col - 1.0 + off.astype(jnp.float32)
    pos = jnp.where(pos >= float(B), pos - float(B), pos)
    lane_t = lax.broadcasted_iota(jnp.int32, (B, B), 1).astype(jnp.float32)
    combined = ((pos == lane_t) & (k > 0.0)).astype(jnp.float32)

    def compact(v_row):
        return lax.dot_general(v_row, combined, (((1,), (0,)), ((), ())),
                               precision=lax.Precision.HIGHEST,
                               preferred_element_type=jnp.float32)

    lane = lax.broadcasted_iota(jnp.int32, (1, B), 1)
    hi_mask = lane >= off
    for ref, v_row in ((llx1, x1jr), (lly1, y1jr), (llx2, x2jr), (lly2, y2jr)):
        rolled = compact(v_row)
        cur0 = ref[pl.ds(qc, 1)].reshape(1, B)
        ref[pl.ds(qc, 1)] = jnp.where(hi_mask, rolled, cur0).reshape(1, 1, B)
        cur1 = ref[pl.ds(qc + 1, 1)].reshape(1, B)
        ref[pl.ds(qc + 1, 1)] = jnp.where(
            hi_mask, cur1, rolled).reshape(1, 1, B)
    c_total = jnp.sum(k).astype(jnp.int32)
    qref[0] = q + c_total


def _nms_keep(bp):
    npad, _ = bp.shape
    nb = npad // B
    x1 = bp[:, 0].reshape(nb, 1, B)
    y1 = bp[:, 1].reshape(nb, 1, B)
    x2 = bp[:, 2].reshape(nb, 1, B)
    y2 = bp[:, 3].reshape(nb, 1, B)
    eye = jnp.eye(B, dtype=jnp.float32)
    tru = jnp.triu(jnp.ones((B, B), jnp.float32))
    full_r = pl.BlockSpec((nb, 1, B), lambda j: (0, 0, 0))
    full_e = pl.BlockSpec((B, B), lambda j: (0, 0))
    keep = pl.pallas_call(
        functools.partial(_nms_body, nb=nb),
        grid=(nb,),
        in_specs=[full_r, full_r, full_r, full_r, full_e, full_e],
        out_specs=full_r,
        out_shape=jax.ShapeDtypeStruct((nb, 1, B), jnp.float32),
        scratch_shapes=[pltpu.VMEM((nb + 1, 1, B), jnp.float32)
                        for _ in range(4)]
        + [pltpu.SMEM((1,), jnp.int32)],
    )(x1, y1, x2, y2, eye, tru)
    return keep.reshape(npad)


_SC_NW = 32
_SC_ROWS_PER_W = 640
_SC_CHUNK = 128
_SC_PAD = _SC_NW * _SC_ROWS_PER_W


def _sc_sorted_gather(table, idx):
    mesh = plsc.VectorSubcoreMesh(core_axis_name="c", subcore_axis_name="s")

    @functools.partial(
        pl.kernel, mesh=mesh,
        out_type=jax.ShapeDtypeStruct((_SC_PAD, 128), jnp.float32),
        scratch_types=[
            pltpu.VMEM((_SC_ROWS_PER_W,), jnp.int32),
            pltpu.VMEM((_SC_ROWS_PER_W, 128), jnp.float32),
            pltpu.SemaphoreType.DMA,
        ],
    )
    def gather_k(table_hbm, idx_hbm, out_hbm, idx_v, rows_v, sem):
        wid = lax.axis_index("s") * 2 + lax.axis_index("c")
        base = wid * _SC_ROWS_PER_W
        pltpu.sync_copy(idx_hbm.at[pl.ds(base, _SC_ROWS_PER_W)], idx_v)
        copies = [
            pltpu.async_copy(
                table_hbm.at[idx_v.at[pl.ds(c * _SC_CHUNK, _SC_CHUNK)]],
                rows_v.at[pl.ds(c * _SC_CHUNK, _SC_CHUNK)], sem)
            for c in range(_SC_ROWS_PER_W // _SC_CHUNK)
        ]
        for cp in copies:
            cp.wait()
        pltpu.sync_copy(rows_v, out_hbm.at[pl.ds(base, _SC_ROWS_PER_W)])

    return gather_k(table, idx)


def kernel(boxes, scores):
    n = boxes.shape[0]
    order = jnp.argsort(-scores)
    table = jnp.concatenate(
        [boxes, scores[:, None], jnp.zeros((n, 123), jnp.float32)], axis=1)
    idx = jnp.concatenate(
        [order.astype(jnp.int32),
         jnp.zeros((_SC_PAD - n,), jnp.int32)])
    sorted_rows = _sc_sorted_gather(table, idx)
    b = sorted_rows[:n, :4]
    s = sorted_rows[:n, 4]
    npad = ((n + B - 1) // B) * B
    pad = jnp.tile(jnp.array([[0.0, 0.0, -1.0, -1.0]], jnp.float32),
                   (npad - n, 1))
    bp = jnp.concatenate([b, pad], axis=0)
    keep = _nms_keep(bp)[:n]
    out = jnp.concatenate([b * keep[:, None], (s * keep)[:, None]], axis=1)
    return out
